# baseline (speedup 1.0000x reference)
import functools

import jax
import jax.numpy as jnp
from jax import lax
from jax.experimental import pallas as pl
from jax.experimental.pallas import tpu as pltpu
from jax.experimental.pallas import tpu_sc as plsc

_B, _S, _D = 4, 4096, 2048
_N = _B * _S
_IDX = 3
_NW = 32
_CHUNK = _N // _NW
_L = 16


def _sc_select(input_):
    mesh = plsc.VectorSubcoreMesh(core_axis_name="c", subcore_axis_name="s")

    @functools.partial(
        pl.kernel,
        mesh=mesh,
        out_type=jax.ShapeDtypeStruct((_B, _S, 1), jnp.float32),
        scratch_types=[
            pltpu.VMEM((_CHUNK, 128), jnp.float32),
            pltpu.VMEM((_CHUNK, 1), jnp.float32),
        ],
        compiler_params=pltpu.CompilerParams(needs_layout_passes=False),
    )
    def k(in_hbm, out_hbm, slab, outb):
        wid = lax.axis_index("s") * 2 + lax.axis_index("c")
        base = wid * _CHUNK
        b = base // _S
        s0 = base % _S
        pltpu.sync_copy(in_hbm.at[b, pl.ds(s0, _CHUNK), pl.ds(0, 128)], slab)
        col = jnp.full((_L,), _IDX, jnp.int32)
        zeros = jnp.zeros((_L,), jnp.int32)

        def body(j, _):
            rows = lax.iota(jnp.int32, _L) + (j * _L)
            vals = plsc.load_gather(slab, [rows, col])
            plsc.store_scatter(outb, [rows, zeros], vals)
            return _

        lax.fori_loop(0, _CHUNK // _L, body, 0)
        pltpu.sync_copy(outb, out_hbm.at[b, pl.ds(s0, _CHUNK), pl.ds(0, 1)])

    return k(input_)


def kernel(input_):
    return _sc_select(input_)


# speedup vs baseline: 1.9651x; 1.9651x over previous
import functools

import jax
import jax.numpy as jnp
from jax.experimental import pallas as pl
from jax.experimental.pallas import tpu as pltpu

_B, _S, _D = 4, 4096, 2048
_IDX = 3
_BLK = 2048


def _tc_body(in_ref, out_ref):
    onehot = jnp.where(jax.lax.broadcasted_iota(jnp.int32, (128, 1), 0) == _IDX,
                       jnp.float32(1), jnp.float32(0))
    out_ref[0, :, :] = jax.lax.dot_general(
        in_ref[0, :, :], onehot,
        dimension_numbers=(((1,), (0,)), ((), ())),
        preferred_element_type=jnp.float32,
    )


def kernel(input_):
    grid = (_B, _S // _BLK)
    return pl.pallas_call(
        _tc_body,
        grid=grid,
        in_specs=[
            pl.BlockSpec((1, _BLK, 128), lambda b, s: (b, s, 0)),
        ],
        out_specs=pl.BlockSpec((1, _BLK, 1), lambda b, s: (b, s, 0)),
        out_shape=jax.ShapeDtypeStruct((_B, _S, 1), jnp.float32),
    )(input_)
